# baseline (speedup 1.0000x reference)
import functools

import jax
import jax.numpy as jnp
from jax import lax
from jax.experimental import pallas as pl
from jax.experimental.pallas import tpu as pltpu
from jax.experimental.pallas import tpu_sc as plsc

NUM_CORES = 2
NUM_SUBCORES = 16
NUM_WORKERS = NUM_CORES * NUM_SUBCORES
CHUNK = 64
SB = 8
NPAD = 8
ZCHUNK = 400


def _sc_agg(h, ei4, zeros):
    n, d = h.shape
    n -= NPAD
    steps = ei4.shape[2]
    nsb = steps // SB
    n_zchunks = n // ZCHUNK
    assert steps % SB == 0 and SB % 4 == 0 and nsb % 2 == 0 and nsb >= 4

    @functools.partial(
        pl.kernel,
        out_type=jax.ShapeDtypeStruct((NUM_CORES, n, d), jnp.float32),
        mesh=plsc.VectorSubcoreMesh(core_axis_name="c", subcore_axis_name="s"),
        scratch_types=[
            pltpu.VMEM((SB, CHUNK), jnp.int32),
            pltpu.VMEM((SB, CHUNK), jnp.int32),
            pltpu.VMEM((SB, CHUNK), jnp.int32),
            pltpu.VMEM((SB, CHUNK), jnp.int32),
            pltpu.VMEM((CHUNK, d), jnp.float32),
            pltpu.VMEM((CHUNK, d), jnp.float32),
            pltpu.VMEM((CHUNK, d), jnp.float32),
            pltpu.VMEM((CHUNK, d), jnp.float32),
            pltpu.VMEM_SHARED((n, d), jnp.float32),
            pltpu.SemaphoreType.DMA((4,)),
            pltpu.SemaphoreType.DMA((4,)),
            pltpu.SemaphoreType.DMA((2,)),
            pltpu.SemaphoreType.DMA((2,)),
        ],
    )
    def k(h_hbm, ei_hbm, z_hbm, out_hbm,
          si0, si1, di0, di1, r0, r1, r2, r3, acc, gsem, ssem, xsem, dsem):
        cid = lax.axis_index("c")
        sid = lax.axis_index("s")
        wid = cid * NUM_SUBCORES + sid
        rows = [r0, r1, r2, r3]
        sidx = [si0, si1]
        didx = [di0, di1]

        def issue_idx(j, p):
            pltpu.async_copy(ei_hbm.at[0, wid, pl.ds(j * SB, SB), :],
                             sidx[p], xsem.at[p])
            pltpu.async_copy(ei_hbm.at[1, wid, pl.ds(j * SB, SB), :],
                             didx[p], dsem.at[p])

        def wait_idx(p):
            pltpu.make_async_copy(ei_hbm.at[0, wid, pl.ds(0, SB), :],
                                  sidx[p], xsem.at[p]).wait()
            pltpu.make_async_copy(ei_hbm.at[1, wid, pl.ds(0, SB), :],
                                  didx[p], dsem.at[p]).wait()

        def issue_gather(p, t, b):
            pltpu.async_copy(h_hbm.at[sidx[p].at[t]], rows[b], gsem.at[b])

        def wait_gather(b):
            pltpu.make_async_copy(h_hbm.at[sidx[0].at[0]], rows[b],
                                  gsem.at[b]).wait()

        def issue_scatter(p, t, b):
            pltpu.async_copy(rows[b], acc.at[didx[p].at[t]], ssem.at[b],
                             add=True)

        def wait_scatter(b):
            pltpu.make_async_copy(rows[b], acc.at[didx[0].at[0]],
                                  ssem.at[b]).wait()

        def guard(cond, fn):
            if isinstance(cond, bool):
                if cond:
                    fn()
            else:
                pl.when(cond)(fn)

        def sb_body(j, p, not_last, not_penult, first=False):
            for t in range(SB):
                b = t % 4
                wait_gather(b)
                issue_scatter(p, t, b)
                if not (first and t < 2):
                    wait_scatter((b + 2) % 4)
                if t < SB - 2:
                    issue_gather(p, t + 2, (t + 2) % 4)
                elif t == SB - 2:
                    def _pref0():
                        wait_idx(1 - p)
                        issue_gather(1 - p, 0, (t + 2) % 4)
                    guard(not_last, _pref0)
                else:
                    def _pref1():
                        issue_gather(1 - p, 1, (t + 2) % 4)
                    guard(not_last, _pref1)

                    def _nexti():
                        issue_idx(j + 2, p)
                    guard(not_penult, _nexti)

        @pl.loop(sid, n_zchunks, step=NUM_SUBCORES)
        def _(z):
            pltpu.sync_copy(z_hbm, acc.at[pl.ds(z * ZCHUNK, ZCHUNK), :])

        plsc.subcore_barrier()

        issue_idx(0, 0)
        wait_idx(0)
        issue_idx(1, 1)
        issue_gather(0, 0, 0)
        issue_gather(0, 1, 1)

        sb_body(0, 0, True, True, first=True)
        sb_body(1, 1, True, True)

        @pl.loop(1, nsb // 2)
        def _(jj):
            sb_body(2 * jj, 0, True, jj < (nsb // 2 - 1))
            sb_body(2 * jj + 1, 1, jj < (nsb // 2 - 1), jj < (nsb // 2 - 1))

        wait_scatter(2)
        wait_scatter(3)

        plsc.subcore_barrier()

        @pl.loop(sid, n_zchunks, step=NUM_SUBCORES)
        def _(z):
            pltpu.sync_copy(acc.at[pl.ds(z * ZCHUNK, ZCHUNK), :],
                            out_hbm.at[cid, pl.ds(z * ZCHUNK, ZCHUNK), :])

    return k(h, ei4, zeros)


def _mlp(x, p, W1, b1, W2, b2, relu_out, block):
    n, d = x.shape

    def body(x_ref, p0_ref, p1_ref, w1_ref, b1_ref, w2_ref, b2_ref, o_ref):
        h = x_ref[...] + p0_ref[...] + p1_ref[...]
        t = jnp.dot(h, w1_ref[...], preferred_element_type=jnp.float32)
        t = jnp.maximum(t + b1_ref[...], 0.0)
        o = jnp.dot(t, w2_ref[...], preferred_element_type=jnp.float32)
        o = o + b2_ref[...]
        if relu_out:
            o = jnp.maximum(o, 0.0)
        o_ref[...] = o

    row_spec = pl.BlockSpec((block, d), lambda i: (i, 0))
    full_mat = pl.BlockSpec((d, d), lambda i: (0, 0))
    full_vec = pl.BlockSpec((1, d), lambda i: (0, 0))
    return pl.pallas_call(
        body,
        grid=(n // block,),
        in_specs=[row_spec, row_spec, row_spec,
                  full_mat, full_vec, full_mat, full_vec],
        out_specs=row_spec,
        out_shape=jax.ShapeDtypeStruct((n, d), jnp.float32),
    )(x, p[0], p[1], W1, b1.reshape(1, d), W2, b2.reshape(1, d))


def kernel(x, edge_index,
           W1_0, b1_0, W2_0, b2_0,
           W1_1, b1_1, W2_1, b2_1,
           W1_2, b1_2, W2_2, b2_2):
    n, d = x.shape
    e = edge_index.shape[1]
    steps = -(-e // (NUM_WORKERS * CHUNK))
    if steps % (2 * SB) != 0:
        steps = -(-steps // (2 * SB)) * (2 * SB)
    e_pad = NUM_WORKERS * steps * CHUNK
    pad = e_pad - e
    if pad:
        dummy = jnp.stack([jnp.full((pad,), n, jnp.int32),
                           jnp.arange(pad, dtype=jnp.int32) % n])
        ei = jnp.concatenate([edge_index, dummy], axis=1)
    else:
        ei = edge_index
    ei4 = ei.reshape(2, NUM_WORKERS, steps, CHUNK)
    zeros = jnp.zeros((ZCHUNK, d), jnp.float32)
    hpad = jnp.zeros((NPAD, d), jnp.float32)

    h = x
    for i, (W1, b1, W2, b2) in enumerate([
            (W1_0, b1_0, W2_0, b2_0),
            (W1_1, b1_1, W2_1, b2_1),
            (W1_2, b1_2, W2_2, b2_2)]):
        p = _sc_agg(jnp.concatenate([h, hpad]), ei4, zeros)
        h = _mlp(h, p, W1, b1, W2, b2, relu_out=(i < 2), block=1000)
    return h


# speedup vs baseline: 3.7885x; 3.7885x over previous
import functools

import jax
import jax.numpy as jnp
from jax import lax
from jax.experimental import pallas as pl
from jax.experimental.pallas import tpu as pltpu
from jax.experimental.pallas import tpu_sc as plsc

NUM_CORES = 2
NUM_SUBCORES = 16
NUM_WORKERS = NUM_CORES * NUM_SUBCORES
CHUNK = 64
SB = 8
NPAD = 64
ZCHUNK = 400


def _sc_agg(h, ei4, zeros):
    n, d = h.shape
    n -= NPAD
    steps = ei4.shape[2]
    nsb = steps // SB
    n_zchunks = n // ZCHUNK
    assert steps % SB == 0 and SB % 4 == 0 and nsb % 2 == 0 and nsb >= 4

    @functools.partial(
        pl.kernel,
        out_type=jax.ShapeDtypeStruct((NUM_CORES, n, d), jnp.float32),
        mesh=plsc.VectorSubcoreMesh(core_axis_name="c", subcore_axis_name="s"),
        scratch_types=[
            pltpu.VMEM((SB, CHUNK), jnp.int32),
            pltpu.VMEM((SB, CHUNK), jnp.int32),
            pltpu.VMEM((SB, CHUNK), jnp.int32),
            pltpu.VMEM((SB, CHUNK), jnp.int32),
            pltpu.VMEM((CHUNK, d), jnp.float32),
            pltpu.VMEM((CHUNK, d), jnp.float32),
            pltpu.VMEM((CHUNK, d), jnp.float32),
            pltpu.VMEM((CHUNK, d), jnp.float32),
            pltpu.VMEM_SHARED((n, d), jnp.float32),
            pltpu.SemaphoreType.DMA((4,)),
            pltpu.SemaphoreType.DMA((4,)),
            pltpu.SemaphoreType.DMA((2,)),
            pltpu.SemaphoreType.DMA((2,)),
        ],
    )
    def k(h_hbm, ei_hbm, z_hbm, out_hbm,
          si0, si1, di0, di1, r0, r1, r2, r3, acc, gsem, ssem, xsem, dsem):
        cid = lax.axis_index("c")
        sid = lax.axis_index("s")
        wid = cid * NUM_SUBCORES + sid
        rows = [r0, r1, r2, r3]
        sidx = [si0, si1]
        didx = [di0, di1]

        def issue_idx(j, p):
            pltpu.async_copy(ei_hbm.at[0, wid, pl.ds(j * SB, SB), :],
                             sidx[p], xsem.at[p])
            pltpu.async_copy(ei_hbm.at[1, wid, pl.ds(j * SB, SB), :],
                             didx[p], dsem.at[p])

        def wait_idx(p):
            pltpu.make_async_copy(ei_hbm.at[0, wid, pl.ds(0, SB), :],
                                  sidx[p], xsem.at[p]).wait()
            pltpu.make_async_copy(ei_hbm.at[1, wid, pl.ds(0, SB), :],
                                  didx[p], dsem.at[p]).wait()

        def issue_gather(p, t, b):
            pltpu.async_copy(h_hbm.at[sidx[p].at[t]], rows[b], gsem.at[b])

        def wait_gather(b):
            pltpu.make_async_copy(h_hbm.at[sidx[0].at[0]], rows[b],
                                  gsem.at[b]).wait()

        def issue_scatter(p, t, b):
            pltpu.async_copy(rows[b], acc.at[didx[p].at[t]], ssem.at[b],
                             add=True)

        def wait_scatter(b):
            pltpu.make_async_copy(rows[b], acc.at[didx[0].at[0]],
                                  ssem.at[b]).wait()

        def guard(cond, fn):
            if isinstance(cond, bool):
                if cond:
                    fn()
            else:
                pl.when(cond)(fn)

        def sb_body(j, p, not_last, not_penult, first=False):
            for t in range(SB):
                b = t % 4
                wait_gather(b)
                issue_scatter(p, t, b)
                if not (first and t < 2):
                    wait_scatter((b + 2) % 4)
                if t < SB - 2:
                    issue_gather(p, t + 2, (t + 2) % 4)
                elif t == SB - 2:
                    def _pref0():
                        wait_idx(1 - p)
                        issue_gather(1 - p, 0, (t + 2) % 4)
                    guard(not_last, _pref0)
                else:
                    def _pref1():
                        issue_gather(1 - p, 1, (t + 2) % 4)
                    guard(not_last, _pref1)

                    def _nexti():
                        issue_idx(j + 2, p)
                    guard(not_penult, _nexti)

        @pl.loop(sid, n_zchunks, step=NUM_SUBCORES)
        def _(z):
            pltpu.sync_copy(z_hbm, acc.at[pl.ds(z * ZCHUNK, ZCHUNK), :])

        plsc.subcore_barrier()

        issue_idx(0, 0)
        wait_idx(0)
        issue_idx(1, 1)
        issue_gather(0, 0, 0)
        issue_gather(0, 1, 1)

        sb_body(0, 0, True, True, first=True)
        sb_body(1, 1, True, True)

        @pl.loop(1, nsb // 2)
        def _(jj):
            sb_body(2 * jj, 0, True, jj < (nsb // 2 - 1))
            sb_body(2 * jj + 1, 1, jj < (nsb // 2 - 1), jj < (nsb // 2 - 1))

        wait_scatter(2)
        wait_scatter(3)

        plsc.subcore_barrier()

        @pl.loop(sid, n_zchunks, step=NUM_SUBCORES)
        def _(z):
            pltpu.sync_copy(acc.at[pl.ds(z * ZCHUNK, ZCHUNK), :],
                            out_hbm.at[cid, pl.ds(z * ZCHUNK, ZCHUNK), :])

    return k(h, ei4, zeros)


def _mlp(x, p, W1, b1, W2, b2, relu_out, block):
    n, d = x.shape

    def body(x_ref, p0_ref, p1_ref, w1_ref, b1_ref, w2_ref, b2_ref, o_ref):
        h = x_ref[...] + p0_ref[...] + p1_ref[...]
        t = jnp.dot(h, w1_ref[...], preferred_element_type=jnp.float32)
        t = jnp.maximum(t + b1_ref[...], 0.0)
        o = jnp.dot(t, w2_ref[...], preferred_element_type=jnp.float32)
        o = o + b2_ref[...]
        if relu_out:
            o = jnp.maximum(o, 0.0)
        o_ref[...] = o

    row_spec = pl.BlockSpec((block, d), lambda i: (i, 0))
    full_mat = pl.BlockSpec((d, d), lambda i: (0, 0))
    full_vec = pl.BlockSpec((1, d), lambda i: (0, 0))
    return pl.pallas_call(
        body,
        grid=(n // block,),
        in_specs=[row_spec, row_spec, row_spec,
                  full_mat, full_vec, full_mat, full_vec],
        out_specs=row_spec,
        out_shape=jax.ShapeDtypeStruct((n, d), jnp.float32),
    )(x, p[0], p[1], W1, b1.reshape(1, d), W2, b2.reshape(1, d))


def kernel(x, edge_index,
           W1_0, b1_0, W2_0, b2_0,
           W1_1, b1_1, W2_1, b2_1,
           W1_2, b1_2, W2_2, b2_2):
    n, d = x.shape
    e = edge_index.shape[1]
    steps = -(-e // (NUM_WORKERS * CHUNK))
    if steps % (2 * SB) != 0:
        steps = -(-steps // (2 * SB)) * (2 * SB)
    e_pad = NUM_WORKERS * steps * CHUNK
    pad = e_pad - e
    if pad:
        ar = jnp.arange(pad, dtype=jnp.int32)
        dummy = jnp.stack([n + ar % NPAD, ar % n])
        ei = jnp.concatenate([edge_index, dummy], axis=1)
    else:
        ei = edge_index
    ei4 = ei.reshape(2, NUM_WORKERS, steps, CHUNK)
    zeros = jnp.zeros((ZCHUNK, d), jnp.float32)
    hpad = jnp.zeros((NPAD, d), jnp.float32)

    h = x
    for i, (W1, b1, W2, b2) in enumerate([
            (W1_0, b1_0, W2_0, b2_0),
            (W1_1, b1_1, W2_1, b2_1),
            (W1_2, b1_2, W2_2, b2_2)]):
        p = _sc_agg(jnp.concatenate([h, hpad]), ei4, zeros)
        h = _mlp(h, p, W1, b1, W2, b2, relu_out=(i < 2), block=1000)
    return h


# speedup vs baseline: 4.0288x; 1.0634x over previous
import functools

import jax
import jax.numpy as jnp
from jax import lax
from jax.experimental import pallas as pl
from jax.experimental.pallas import tpu as pltpu
from jax.experimental.pallas import tpu_sc as plsc

NUM_CORES = 2
NUM_SUBCORES = 16
NUM_WORKERS = NUM_CORES * NUM_SUBCORES
CHUNK = 64
SB = 8
NPAD = 64
ZCHUNK = 400


def _sc_agg(h, ei4, zeros):
    n, d = h.shape
    n -= NPAD
    steps = ei4.shape[2]
    nsb = steps // SB
    n_zchunks = n // ZCHUNK
    assert steps % SB == 0 and SB % 4 == 0 and nsb % 2 == 0 and nsb >= 4

    @functools.partial(
        pl.kernel,
        out_type=jax.ShapeDtypeStruct((NUM_CORES, n, d), jnp.float32),
        mesh=plsc.VectorSubcoreMesh(core_axis_name="c", subcore_axis_name="s"),
        scratch_types=[
            pltpu.VMEM((SB, CHUNK), jnp.int32),
            pltpu.VMEM((SB, CHUNK), jnp.int32),
            pltpu.VMEM((SB, CHUNK), jnp.int32),
            pltpu.VMEM((SB, CHUNK), jnp.int32),
            pltpu.VMEM((CHUNK, d), jnp.float32),
            pltpu.VMEM((CHUNK, d), jnp.float32),
            pltpu.VMEM((CHUNK, d), jnp.float32),
            pltpu.VMEM((CHUNK, d), jnp.float32),
            pltpu.VMEM_SHARED((n + NPAD, d), jnp.float32),
            pltpu.SemaphoreType.DMA((4,)),
            pltpu.SemaphoreType.DMA((4,)),
            pltpu.SemaphoreType.DMA((2,)),
            pltpu.SemaphoreType.DMA((2,)),
        ],
    )
    def k(h_hbm, ei_hbm, z_hbm, out_hbm,
          si0, si1, di0, di1, r0, r1, r2, r3, acc, gsem, ssem, xsem, dsem):
        cid = lax.axis_index("c")
        sid = lax.axis_index("s")
        wid = cid * NUM_SUBCORES + sid
        rows = [r0, r1, r2, r3]
        sidx = [si0, si1]
        didx = [di0, di1]

        def issue_idx(j, p):
            pltpu.async_copy(ei_hbm.at[0, wid, pl.ds(j * SB, SB), :],
                             sidx[p], xsem.at[p])
            pltpu.async_copy(ei_hbm.at[1, wid, pl.ds(j * SB, SB), :],
                             didx[p], dsem.at[p])

        def wait_idx(p):
            pltpu.make_async_copy(ei_hbm.at[0, wid, pl.ds(0, SB), :],
                                  sidx[p], xsem.at[p]).wait()
            pltpu.make_async_copy(ei_hbm.at[1, wid, pl.ds(0, SB), :],
                                  didx[p], dsem.at[p]).wait()

        def issue_gather(p, t, b):
            pltpu.async_copy(h_hbm.at[sidx[p].at[t]], rows[b], gsem.at[b])

        def wait_gather(b):
            pltpu.make_async_copy(h_hbm.at[sidx[0].at[0]], rows[b],
                                  gsem.at[b]).wait()

        def issue_scatter(p, t, b):
            pltpu.async_copy(rows[b], acc.at[didx[p].at[t]], ssem.at[b],
                             add=True)

        def wait_scatter(b):
            pltpu.make_async_copy(rows[b], acc.at[didx[0].at[0]],
                                  ssem.at[b]).wait()

        def guard(cond, fn):
            if isinstance(cond, bool):
                if cond:
                    fn()
            else:
                pl.when(cond)(fn)

        def sb_body(j, p, not_last, not_penult, first=False):
            for t in range(SB):
                b = t % 4
                wait_gather(b)
                issue_scatter(p, t, b)
                if not (first and t < 2):
                    wait_scatter((b + 2) % 4)
                if t < SB - 2:
                    issue_gather(p, t + 2, (t + 2) % 4)
                elif t == SB - 2:
                    def _pref0():
                        wait_idx(1 - p)
                        issue_gather(1 - p, 0, (t + 2) % 4)
                    guard(not_last, _pref0)
                else:
                    def _pref1():
                        issue_gather(1 - p, 1, (t + 2) % 4)
                    guard(not_last, _pref1)

                    def _nexti():
                        issue_idx(j + 2, p)
                    guard(not_penult, _nexti)

        @pl.loop(sid, n_zchunks, step=NUM_SUBCORES)
        def _(z):
            pltpu.sync_copy(z_hbm, acc.at[pl.ds(z * ZCHUNK, ZCHUNK), :])

        plsc.subcore_barrier()

        issue_idx(0, 0)
        wait_idx(0)
        issue_idx(1, 1)
        issue_gather(0, 0, 0)
        issue_gather(0, 1, 1)

        sb_body(0, 0, True, True, first=True)
        sb_body(1, 1, True, True)

        @pl.loop(1, nsb // 2)
        def _(jj):
            sb_body(2 * jj, 0, True, jj < (nsb // 2 - 1))
            sb_body(2 * jj + 1, 1, jj < (nsb // 2 - 1), jj < (nsb // 2 - 1))

        wait_scatter(2)
        wait_scatter(3)

        plsc.subcore_barrier()

        @pl.loop(sid, n_zchunks, step=NUM_SUBCORES)
        def _(z):
            pltpu.sync_copy(acc.at[pl.ds(z * ZCHUNK, ZCHUNK), :],
                            out_hbm.at[cid, pl.ds(z * ZCHUNK, ZCHUNK), :])

    return k(h, ei4, zeros)


def _mlp(x, p, W1, b1, W2, b2, relu_out, block, n_out):
    d = x.shape[1]
    n = p.shape[1]

    def body(x_ref, p0_ref, p1_ref, w1_ref, b1_ref, w2_ref, b2_ref, o_ref):
        h = x_ref[...] + p0_ref[0] + p1_ref[0]
        t = jnp.dot(h, w1_ref[...], preferred_element_type=jnp.float32)
        t = jnp.maximum(t + b1_ref[...], 0.0)
        o = jnp.dot(t, w2_ref[...], preferred_element_type=jnp.float32)
        o = o + b2_ref[...]
        if relu_out:
            o = jnp.maximum(o, 0.0)
        o_ref[...] = o

    row_spec = pl.BlockSpec((block, d), lambda i: (i, 0))
    p0_spec = pl.BlockSpec((1, block, d), lambda i: (0, i, 0))
    p1_spec = pl.BlockSpec((1, block, d), lambda i: (1, i, 0))
    full_mat = pl.BlockSpec((d, d), lambda i: (0, 0))
    full_vec = pl.BlockSpec((1, d), lambda i: (0, 0))
    return pl.pallas_call(
        body,
        grid=(-(-n_out // block),),
        in_specs=[row_spec, p0_spec, p1_spec,
                  full_mat, full_vec, full_mat, full_vec],
        out_specs=row_spec,
        out_shape=jax.ShapeDtypeStruct((n_out, d), jnp.float32),
    )(x, p, p, W1, b1.reshape(1, d), W2, b2.reshape(1, d))


def kernel(x, edge_index,
           W1_0, b1_0, W2_0, b2_0,
           W1_1, b1_1, W2_1, b2_1,
           W1_2, b1_2, W2_2, b2_2):
    n, d = x.shape
    e = edge_index.shape[1]
    steps = -(-e // (NUM_WORKERS * CHUNK))
    if steps % (2 * SB) != 0:
        steps = -(-steps // (2 * SB)) * (2 * SB)
    e_pad = NUM_WORKERS * steps * CHUNK
    pad = e_pad - e
    if pad:
        ar = jnp.arange(pad, dtype=jnp.int32)
        dummy = jnp.stack([n + ar % NPAD, n + ar % NPAD])
        ei = jnp.concatenate([edge_index, dummy], axis=1)
    else:
        ei = edge_index
    ei4 = ei.reshape(2, NUM_WORKERS, steps, CHUNK)
    zeros = jnp.zeros((ZCHUNK, d), jnp.float32)

    h = jnp.concatenate([x, jnp.zeros((NPAD, d), jnp.float32)])
    for i, (W1, b1, W2, b2) in enumerate([
            (W1_0, b1_0, W2_0, b2_0),
            (W1_1, b1_1, W2_1, b2_1),
            (W1_2, b1_2, W2_2, b2_2)]):
        p = _sc_agg(h, ei4, zeros)
        n_out = n if i == 2 else n + NPAD
        h = _mlp(h, p, W1, b1, W2, b2, relu_out=(i < 2), block=1000,
                 n_out=n_out)
    return h


# speedup vs baseline: 4.1764x; 1.0366x over previous
import functools

import jax
import jax.numpy as jnp
from jax import lax
from jax.experimental import pallas as pl
from jax.experimental.pallas import tpu as pltpu
from jax.experimental.pallas import tpu_sc as plsc

NUM_CORES = 2
NUM_SUBCORES = 16
NUM_WORKERS = NUM_CORES * NUM_SUBCORES
CHUNK = 80
SB = 8
NPAD = 128
ZCHUNK = 400


def _sc_agg(h, ei4, zeros):
    n, d = h.shape
    n -= NPAD
    steps = ei4.shape[2]
    nsb = steps // SB
    n_zchunks = n // ZCHUNK
    assert steps % SB == 0 and SB % 4 == 0 and nsb % 2 == 0 and nsb >= 4

    @functools.partial(
        pl.kernel,
        out_type=jax.ShapeDtypeStruct((NUM_CORES, n, d), jnp.float32),
        mesh=plsc.VectorSubcoreMesh(core_axis_name="c", subcore_axis_name="s"),
        scratch_types=[
            pltpu.VMEM((SB, CHUNK), jnp.int32),
            pltpu.VMEM((SB, CHUNK), jnp.int32),
            pltpu.VMEM((SB, CHUNK), jnp.int32),
            pltpu.VMEM((SB, CHUNK), jnp.int32),
            pltpu.VMEM((CHUNK, d), jnp.float32),
            pltpu.VMEM((CHUNK, d), jnp.float32),
            pltpu.VMEM((CHUNK, d), jnp.float32),
            pltpu.VMEM((CHUNK, d), jnp.float32),
            pltpu.VMEM_SHARED((n + NPAD, d), jnp.float32),
            pltpu.SemaphoreType.DMA((4,)),
            pltpu.SemaphoreType.DMA((4,)),
            pltpu.SemaphoreType.DMA((2,)),
            pltpu.SemaphoreType.DMA((2,)),
        ],
    )
    def k(h_hbm, ei_hbm, z_hbm, out_hbm,
          si0, si1, di0, di1, r0, r1, r2, r3, acc, gsem, ssem, xsem, dsem):
        cid = lax.axis_index("c")
        sid = lax.axis_index("s")
        wid = cid * NUM_SUBCORES + sid
        rows = [r0, r1, r2, r3]
        sidx = [si0, si1]
        didx = [di0, di1]

        def issue_idx(j, p):
            pltpu.async_copy(ei_hbm.at[0, wid, pl.ds(j * SB, SB), :],
                             sidx[p], xsem.at[p])
            pltpu.async_copy(ei_hbm.at[1, wid, pl.ds(j * SB, SB), :],
                             didx[p], dsem.at[p])

        def wait_idx(p):
            pltpu.make_async_copy(ei_hbm.at[0, wid, pl.ds(0, SB), :],
                                  sidx[p], xsem.at[p]).wait()
            pltpu.make_async_copy(ei_hbm.at[1, wid, pl.ds(0, SB), :],
                                  didx[p], dsem.at[p]).wait()

        def issue_gather(p, t, b):
            pltpu.async_copy(h_hbm.at[sidx[p].at[t]], rows[b], gsem.at[b])

        def wait_gather(b):
            pltpu.make_async_copy(h_hbm.at[sidx[0].at[0]], rows[b],
                                  gsem.at[b]).wait()

        def issue_scatter(p, t, b):
            pltpu.async_copy(rows[b], acc.at[didx[p].at[t]], ssem.at[b],
                             add=True)

        def wait_scatter(b):
            pltpu.make_async_copy(rows[b], acc.at[didx[0].at[0]],
                                  ssem.at[b]).wait()

        def guard(cond, fn):
            if isinstance(cond, bool):
                if cond:
                    fn()
            else:
                pl.when(cond)(fn)

        def sb_body(j, p, not_last, not_penult, first=False):
            for t in range(SB):
                b = t % 4
                wait_gather(b)
                issue_scatter(p, t, b)
                if not (first and t < 2):
                    wait_scatter((b + 2) % 4)
                if t < SB - 2:
                    issue_gather(p, t + 2, (t + 2) % 4)
                elif t == SB - 2:
                    def _pref0():
                        wait_idx(1 - p)
                        issue_gather(1 - p, 0, (t + 2) % 4)
                    guard(not_last, _pref0)
                else:
                    def _pref1():
                        issue_gather(1 - p, 1, (t + 2) % 4)
                    guard(not_last, _pref1)

                    def _nexti():
                        issue_idx(j + 2, p)
                    guard(not_penult, _nexti)

        @pl.loop(sid, n_zchunks, step=NUM_SUBCORES)
        def _(z):
            pltpu.sync_copy(z_hbm, acc.at[pl.ds(z * ZCHUNK, ZCHUNK), :])

        plsc.subcore_barrier()

        issue_idx(0, 0)
        wait_idx(0)
        issue_idx(1, 1)
        issue_gather(0, 0, 0)
        issue_gather(0, 1, 1)

        sb_body(0, 0, True, True, first=True)
        sb_body(1, 1, True, True)

        @pl.loop(1, nsb // 2)
        def _(jj):
            sb_body(2 * jj, 0, True, jj < (nsb // 2 - 1))
            sb_body(2 * jj + 1, 1, jj < (nsb // 2 - 1), jj < (nsb // 2 - 1))

        wait_scatter(2)
        wait_scatter(3)

        plsc.subcore_barrier()

        @pl.loop(sid, n_zchunks, step=NUM_SUBCORES)
        def _(z):
            pltpu.sync_copy(acc.at[pl.ds(z * ZCHUNK, ZCHUNK), :],
                            out_hbm.at[cid, pl.ds(z * ZCHUNK, ZCHUNK), :])

    return k(h, ei4, zeros)


def _mlp(x, p, W1, b1, W2, b2, relu_out, block, n_out):
    d = x.shape[1]
    n = p.shape[1]

    def body(x_ref, p0_ref, p1_ref, w1_ref, b1_ref, w2_ref, b2_ref, o_ref):
        h = x_ref[...] + p0_ref[0] + p1_ref[0]
        t = jnp.dot(h, w1_ref[...], preferred_element_type=jnp.float32)
        t = jnp.maximum(t + b1_ref[...], 0.0)
        o = jnp.dot(t, w2_ref[...], preferred_element_type=jnp.float32)
        o = o + b2_ref[...]
        if relu_out:
            o = jnp.maximum(o, 0.0)
        o_ref[...] = o

    row_spec = pl.BlockSpec((block, d), lambda i: (i, 0))
    p0_spec = pl.BlockSpec((1, block, d), lambda i: (0, i, 0))
    p1_spec = pl.BlockSpec((1, block, d), lambda i: (1, i, 0))
    full_mat = pl.BlockSpec((d, d), lambda i: (0, 0))
    full_vec = pl.BlockSpec((1, d), lambda i: (0, 0))
    return pl.pallas_call(
        body,
        grid=(-(-n_out // block),),
        in_specs=[row_spec, p0_spec, p1_spec,
                  full_mat, full_vec, full_mat, full_vec],
        out_specs=row_spec,
        out_shape=jax.ShapeDtypeStruct((n_out, d), jnp.float32),
    )(x, p, p, W1, b1.reshape(1, d), W2, b2.reshape(1, d))


def kernel(x, edge_index,
           W1_0, b1_0, W2_0, b2_0,
           W1_1, b1_1, W2_1, b2_1,
           W1_2, b1_2, W2_2, b2_2):
    n, d = x.shape
    e = edge_index.shape[1]
    steps = -(-e // (NUM_WORKERS * CHUNK))
    if steps % (2 * SB) != 0:
        steps = -(-steps // (2 * SB)) * (2 * SB)
    e_pad = NUM_WORKERS * steps * CHUNK
    pad = e_pad - e
    if pad:
        ar = jnp.arange(pad, dtype=jnp.int32)
        dummy = jnp.stack([n + ar % NPAD, n + ar % NPAD])
        ei = jnp.concatenate([edge_index, dummy], axis=1)
    else:
        ei = edge_index
    ei4 = ei.reshape(2, NUM_WORKERS, steps, CHUNK)
    zeros = jnp.zeros((ZCHUNK, d), jnp.float32)

    h = jnp.concatenate([x, jnp.zeros((NPAD, d), jnp.float32)])
    for i, (W1, b1, W2, b2) in enumerate([
            (W1_0, b1_0, W2_0, b2_0),
            (W1_1, b1_1, W2_1, b2_1),
            (W1_2, b1_2, W2_2, b2_2)]):
        p = _sc_agg(h, ei4, zeros)
        n_out = n if i == 2 else n + NPAD
        h = _mlp(h, p, W1, b1, W2, b2, relu_out=(i < 2), block=1000,
                 n_out=n_out)
    return h


# speedup vs baseline: 4.2230x; 1.0112x over previous
import functools

import numpy as np

import jax
import jax.numpy as jnp
from jax import lax
from jax.experimental import pallas as pl
from jax.experimental.pallas import tpu as pltpu
from jax.experimental.pallas import tpu_sc as plsc

NUM_CORES = 2
NUM_SUBCORES = 16
NUM_WORKERS = NUM_CORES * NUM_SUBCORES
CHUNK = 80
SB = 8
NPAD = 128
ZCHUNK = 400


def _sc_agg(h, ei4, zeros):
    n, d = h.shape
    n -= NPAD
    steps = ei4.shape[2]
    nsb = steps // SB
    n_zchunks = n // ZCHUNK
    assert steps % SB == 0 and SB % 4 == 0 and nsb % 2 == 0 and nsb >= 4

    @functools.partial(
        pl.kernel,
        out_type=jax.ShapeDtypeStruct((NUM_CORES, n, d), jnp.float32),
        mesh=plsc.VectorSubcoreMesh(core_axis_name="c", subcore_axis_name="s"),
        scratch_types=[
            pltpu.VMEM((SB, CHUNK), jnp.int32),
            pltpu.VMEM((SB, CHUNK), jnp.int32),
            pltpu.VMEM((SB, CHUNK), jnp.int32),
            pltpu.VMEM((SB, CHUNK), jnp.int32),
            pltpu.VMEM((CHUNK, d), jnp.float32),
            pltpu.VMEM((CHUNK, d), jnp.float32),
            pltpu.VMEM((CHUNK, d), jnp.float32),
            pltpu.VMEM((CHUNK, d), jnp.float32),
            pltpu.VMEM_SHARED((n + NPAD, d), jnp.float32),
            pltpu.SemaphoreType.DMA((4,)),
            pltpu.SemaphoreType.DMA((4,)),
            pltpu.SemaphoreType.DMA((2,)),
            pltpu.SemaphoreType.DMA((2,)),
        ],
    )
    def k(h_hbm, ei_hbm, z_hbm, out_hbm,
          si0, si1, di0, di1, r0, r1, r2, r3, acc, gsem, ssem, xsem, dsem):
        cid = lax.axis_index("c")
        sid = lax.axis_index("s")
        wid = cid * NUM_SUBCORES + sid
        rows = [r0, r1, r2, r3]
        sidx = [si0, si1]
        didx = [di0, di1]

        def issue_idx(j, p):
            pltpu.async_copy(ei_hbm.at[0, wid, pl.ds(j * SB, SB), :],
                             sidx[p], xsem.at[p])
            pltpu.async_copy(ei_hbm.at[1, wid, pl.ds(j * SB, SB), :],
                             didx[p], dsem.at[p])

        def wait_idx(p):
            pltpu.make_async_copy(ei_hbm.at[0, wid, pl.ds(0, SB), :],
                                  sidx[p], xsem.at[p]).wait()
            pltpu.make_async_copy(ei_hbm.at[1, wid, pl.ds(0, SB), :],
                                  didx[p], dsem.at[p]).wait()

        def issue_gather(p, t, b):
            pltpu.async_copy(h_hbm.at[sidx[p].at[t]], rows[b], gsem.at[b])

        def wait_gather(b):
            pltpu.make_async_copy(h_hbm.at[sidx[0].at[0]], rows[b],
                                  gsem.at[b]).wait()

        def issue_scatter(p, t, b):
            pltpu.async_copy(rows[b], acc.at[didx[p].at[t]], ssem.at[b],
                             add=True)

        def wait_scatter(b):
            pltpu.make_async_copy(rows[b], acc.at[didx[0].at[0]],
                                  ssem.at[b]).wait()

        def guard(cond, fn):
            if isinstance(cond, bool):
                if cond:
                    fn()
            else:
                pl.when(cond)(fn)

        def sb_body(j, p, not_last, not_penult, first=False):
            for t in range(SB):
                b = t % 4
                wait_gather(b)
                issue_scatter(p, t, b)
                if not (first and t < 2):
                    wait_scatter((b + 2) % 4)
                if t < SB - 2:
                    issue_gather(p, t + 2, (t + 2) % 4)
                elif t == SB - 2:
                    def _pref0():
                        wait_idx(1 - p)
                        issue_gather(1 - p, 0, (t + 2) % 4)
                    guard(not_last, _pref0)
                else:
                    def _pref1():
                        issue_gather(1 - p, 1, (t + 2) % 4)
                    guard(not_last, _pref1)

                    def _nexti():
                        issue_idx(j + 2, p)
                    guard(not_penult, _nexti)

        issue_idx(0, 0)
        issue_idx(1, 1)

        @pl.loop(sid, n_zchunks, step=NUM_SUBCORES)
        def _(z):
            pltpu.sync_copy(z_hbm, acc.at[pl.ds(z * ZCHUNK, ZCHUNK), :])

        wait_idx(0)
        issue_gather(0, 0, 0)
        issue_gather(0, 1, 1)

        plsc.subcore_barrier()

        sb_body(0, 0, True, True, first=True)
        sb_body(1, 1, True, True)

        @pl.loop(1, nsb // 2)
        def _(jj):
            sb_body(2 * jj, 0, True, jj < (nsb // 2 - 1))
            sb_body(2 * jj + 1, 1, jj < (nsb // 2 - 1), jj < (nsb // 2 - 1))

        wait_scatter(2)
        wait_scatter(3)

        plsc.subcore_barrier()

        @pl.loop(sid, n_zchunks, step=NUM_SUBCORES)
        def _(z):
            pltpu.sync_copy(acc.at[pl.ds(z * ZCHUNK, ZCHUNK), :],
                            out_hbm.at[cid, pl.ds(z * ZCHUNK, ZCHUNK), :])

    return k(h, ei4, zeros)


def _mlp(x, p, W1, b1, W2, b2, relu_out, block, n_out):
    d = x.shape[1]
    n = p.shape[1]

    def body(x_ref, p0_ref, p1_ref, w1_ref, b1_ref, w2_ref, b2_ref, o_ref):
        h = x_ref[...] + p0_ref[0] + p1_ref[0]
        t = jnp.dot(h, w1_ref[...], preferred_element_type=jnp.float32)
        t = jnp.maximum(t + b1_ref[...], 0.0)
        o = jnp.dot(t, w2_ref[...], preferred_element_type=jnp.float32)
        o = o + b2_ref[...]
        if relu_out:
            o = jnp.maximum(o, 0.0)
        o_ref[...] = o

    row_spec = pl.BlockSpec((block, d), lambda i: (i, 0))
    p0_spec = pl.BlockSpec((1, block, d), lambda i: (0, i, 0))
    p1_spec = pl.BlockSpec((1, block, d), lambda i: (1, i, 0))
    full_mat = pl.BlockSpec((d, d), lambda i: (0, 0))
    full_vec = pl.BlockSpec((1, d), lambda i: (0, 0))
    return pl.pallas_call(
        body,
        grid=(-(-n_out // block),),
        in_specs=[row_spec, p0_spec, p1_spec,
                  full_mat, full_vec, full_mat, full_vec],
        out_specs=row_spec,
        out_shape=jax.ShapeDtypeStruct((n_out, d), jnp.float32),
    )(x, p, p, W1, b1.reshape(1, d), W2, b2.reshape(1, d))


def kernel(x, edge_index,
           W1_0, b1_0, W2_0, b2_0,
           W1_1, b1_1, W2_1, b2_1,
           W1_2, b1_2, W2_2, b2_2):
    n, d = x.shape
    e = edge_index.shape[1]
    steps = -(-e // (NUM_WORKERS * CHUNK))
    if steps % (2 * SB) != 0:
        steps = -(-steps // (2 * SB)) * (2 * SB)
    e_pad = NUM_WORKERS * steps * CHUNK
    pad = e_pad - e
    if pad:
        ar = np.arange(pad, dtype=np.int32)
        dummy = jnp.asarray(np.stack([n + ar % NPAD, n + ar % NPAD]))
        ei = jnp.concatenate([edge_index, dummy], axis=1)
    else:
        ei = edge_index
    ei4 = ei.reshape(2, NUM_WORKERS, steps, CHUNK)
    zeros = jnp.zeros((ZCHUNK, d), jnp.float32)

    h = jnp.concatenate([x, jnp.zeros((NPAD, d), jnp.float32)])
    for i, (W1, b1, W2, b2) in enumerate([
            (W1_0, b1_0, W2_0, b2_0),
            (W1_1, b1_1, W2_1, b2_1),
            (W1_2, b1_2, W2_2, b2_2)]):
        p = _sc_agg(h, ei4, zeros)
        n_out = n if i == 2 else n + NPAD
        h = _mlp(h, p, W1, b1, W2, b2, relu_out=(i < 2), block=1000,
                 n_out=n_out)
    return h


# speedup vs baseline: 4.3345x; 1.0264x over previous
import functools

import numpy as np

import jax
import jax.numpy as jnp
from jax import lax
from jax.experimental import pallas as pl
from jax.experimental.pallas import tpu as pltpu
from jax.experimental.pallas import tpu_sc as plsc

NUM_CORES = 2
NUM_SUBCORES = 16
NUM_WORKERS = NUM_CORES * NUM_SUBCORES
CHUNK = 80
SB = 8
NPAD = 128
ZCHUNK = 400


def _sc_agg(h, ei4, zeros):
    n, d = h.shape
    n -= NPAD
    steps = ei4.shape[2]
    nsb = steps // SB
    n_zchunks = n // ZCHUNK
    assert steps % SB == 0 and SB % 4 == 0 and nsb % 2 == 0 and nsb >= 4

    @functools.partial(
        pl.kernel,
        out_type=jax.ShapeDtypeStruct((NUM_CORES, n, d), jnp.float32),
        mesh=plsc.VectorSubcoreMesh(core_axis_name="c", subcore_axis_name="s"),
        scratch_types=[
            pltpu.VMEM((SB, CHUNK), jnp.int32),
            pltpu.VMEM((SB, CHUNK), jnp.int32),
            pltpu.VMEM((SB, CHUNK), jnp.int32),
            pltpu.VMEM((SB, CHUNK), jnp.int32),
            pltpu.VMEM((CHUNK, d), jnp.float32),
            pltpu.VMEM((CHUNK, d), jnp.float32),
            pltpu.VMEM((CHUNK, d), jnp.float32),
            pltpu.VMEM((CHUNK, d), jnp.float32),
            pltpu.VMEM_SHARED((n + NPAD, d), jnp.float32),
            pltpu.SemaphoreType.DMA((4,)),
            pltpu.SemaphoreType.DMA((4,)),
            pltpu.SemaphoreType.DMA((2,)),
            pltpu.SemaphoreType.DMA((2,)),
        ],
    )
    def k(h_hbm, ei_hbm, z_hbm, out_hbm,
          si0, si1, di0, di1, r0, r1, r2, r3, acc, gsem, ssem, xsem, dsem):
        cid = lax.axis_index("c")
        sid = lax.axis_index("s")
        wid = cid * NUM_SUBCORES + sid
        rows = [r0, r1, r2, r3]
        sidx = [si0, si1]
        didx = [di0, di1]

        def issue_idx(j, p):
            pltpu.async_copy(ei_hbm.at[0, wid, pl.ds(j * SB, SB), :],
                             sidx[p], xsem.at[p])
            pltpu.async_copy(ei_hbm.at[1, wid, pl.ds(j * SB, SB), :],
                             didx[p], dsem.at[p])

        def wait_idx(p):
            pltpu.make_async_copy(ei_hbm.at[0, wid, pl.ds(0, SB), :],
                                  sidx[p], xsem.at[p]).wait()
            pltpu.make_async_copy(ei_hbm.at[1, wid, pl.ds(0, SB), :],
                                  didx[p], dsem.at[p]).wait()

        def issue_gather(p, t, b):
            pltpu.async_copy(h_hbm.at[sidx[p].at[t]], rows[b], gsem.at[b])

        def wait_gather(b):
            pltpu.make_async_copy(h_hbm.at[sidx[0].at[0]], rows[b],
                                  gsem.at[b]).wait()

        def issue_scatter(p, t, b):
            pltpu.async_copy(rows[b], acc.at[didx[p].at[t]], ssem.at[b],
                             add=True)

        def wait_scatter(b):
            pltpu.make_async_copy(rows[b], acc.at[didx[0].at[0]],
                                  ssem.at[b]).wait()

        def guard(cond, fn):
            if isinstance(cond, bool):
                if cond:
                    fn()
            else:
                pl.when(cond)(fn)

        def sb_body(j, p, not_last, not_penult, first=False):
            for t in range(SB):
                b = t % 4
                wait_gather(b)
                issue_scatter(p, t, b)
                if not (first and t < 2):
                    wait_scatter((b + 2) % 4)
                if t < SB - 2:
                    issue_gather(p, t + 2, (t + 2) % 4)
                elif t == SB - 2:
                    def _pref0():
                        wait_idx(1 - p)
                        issue_gather(1 - p, 0, (t + 2) % 4)
                    guard(not_last, _pref0)
                else:
                    def _pref1():
                        issue_gather(1 - p, 1, (t + 2) % 4)
                    guard(not_last, _pref1)

                    def _nexti():
                        issue_idx(j + 2, p)
                    guard(not_penult, _nexti)

        issue_idx(0, 0)
        issue_idx(1, 1)

        @pl.loop(sid, n_zchunks, step=NUM_SUBCORES)
        def _(z):
            pltpu.sync_copy(z_hbm, acc.at[pl.ds(z * ZCHUNK, ZCHUNK), :])

        wait_idx(0)
        issue_gather(0, 0, 0)
        issue_gather(0, 1, 1)

        plsc.subcore_barrier()

        sb_body(0, 0, True, True, first=True)
        sb_body(1, 1, True, True)

        @pl.loop(1, nsb // 2)
        def _(jj):
            sb_body(2 * jj, 0, True, jj < (nsb // 2 - 1))
            sb_body(2 * jj + 1, 1, jj < (nsb // 2 - 1), jj < (nsb // 2 - 1))

        wait_scatter(2)
        wait_scatter(3)

        plsc.subcore_barrier()

        @pl.loop(sid, n_zchunks, step=NUM_SUBCORES)
        def _(z):
            pltpu.sync_copy(acc.at[pl.ds(z * ZCHUNK, ZCHUNK), :],
                            out_hbm.at[cid, pl.ds(z * ZCHUNK, ZCHUNK), :])

    return k(h, ei4, zeros)


def _mlp(x, p, W1, b1, W2, b2, relu_out, block, n_out):
    d = x.shape[1]
    n = p.shape[1]

    def body(x_ref, p0_ref, p1_ref, w1_ref, b1_ref, w2_ref, b2_ref, o_ref):
        h = x_ref[...] + p0_ref[0] + p1_ref[0]
        t = jnp.dot(h, w1_ref[...], preferred_element_type=jnp.float32)
        t = jnp.maximum(t + b1_ref[...], 0.0)
        o = jnp.dot(t, w2_ref[...], preferred_element_type=jnp.float32)
        o = o + b2_ref[...]
        if relu_out:
            o = jnp.maximum(o, 0.0)
        o_ref[...] = o

    row_spec = pl.BlockSpec((block, d), lambda i: (i, 0))
    p0_spec = pl.BlockSpec((1, block, d), lambda i: (0, i, 0))
    p1_spec = pl.BlockSpec((1, block, d), lambda i: (1, i, 0))
    full_mat = pl.BlockSpec((d, d), lambda i: (0, 0))
    full_vec = pl.BlockSpec((1, d), lambda i: (0, 0))
    return pl.pallas_call(
        body,
        grid=(-(-n_out // block),),
        in_specs=[row_spec, p0_spec, p1_spec,
                  full_mat, full_vec, full_mat, full_vec],
        out_specs=row_spec,
        out_shape=jax.ShapeDtypeStruct((n_out, d), jnp.float32),
    )(x, p, p, W1, b1.reshape(1, d), W2, b2.reshape(1, d))


def kernel(x, edge_index,
           W1_0, b1_0, W2_0, b2_0,
           W1_1, b1_1, W2_1, b2_1,
           W1_2, b1_2, W2_2, b2_2):
    n, d = x.shape
    e = edge_index.shape[1]
    steps = -(-e // (NUM_WORKERS * CHUNK))
    if steps % (2 * SB) != 0:
        steps = -(-steps // (2 * SB)) * (2 * SB)
    e_pad = NUM_WORKERS * steps * CHUNK
    pad = e_pad - e
    if pad:
        ar = np.arange(pad, dtype=np.int32)
        dummy = jnp.asarray(np.stack([n + ar % NPAD, n + ar % NPAD]))
        ei = jnp.concatenate([edge_index, dummy], axis=1)
    else:
        ei = edge_index
    ei4 = ei.reshape(2, NUM_WORKERS, steps, CHUNK)
    zeros = jnp.zeros((ZCHUNK, d), jnp.float32)

    h = jnp.concatenate([x, jnp.zeros((NPAD, d), jnp.float32)])
    for i, (W1, b1, W2, b2) in enumerate([
            (W1_0, b1_0, W2_0, b2_0),
            (W1_1, b1_1, W2_1, b2_1),
            (W1_2, b1_2, W2_2, b2_2)]):
        p = _sc_agg(h, ei4, zeros)
        n_out = n if i == 2 else n + NPAD
        h = _mlp(h, p, W1, b1, W2, b2, relu_out=(i < 2), block=2000,
                 n_out=n_out)
    return h
